# baseline (speedup 1.0000x reference)
import functools

import jax
import jax.numpy as jnp
from jax import lax
from jax.experimental import pallas as pl
from jax.experimental.pallas import tpu as pltpu
from jax.experimental.pallas import tpu_sc as plsc

B = 16384
D = 32
L = 16
NC, NS = 2, 16
NW = NC * NS
RPW = B // NW
CH = 128
NCH = RPW // CH
GROUPS = RPW // L


def _sc_body(uid_hbm, qid_hbm, ut_hbm, qt_hbm, w_hbm, out_hbm,
             idx_u, idx_q, rows_u, rows_q, wv, outbuf, sem):
    c = lax.axis_index("c")
    s = lax.axis_index("s")
    wid = s * NC + c

    pltpu.sync_copy(uid_hbm.at[pl.ds(wid * NCH, NCH)], idx_u)
    pltpu.sync_copy(qid_hbm.at[pl.ds(wid * NCH, NCH)], idx_q)
    pltpu.sync_copy(w_hbm, wv)

    cps = []
    for j in range(NCH):
        cps.append(pltpu.async_copy(
            ut_hbm.at[idx_u.at[j]], rows_u.at[pl.ds(j * CH, CH)], sem))
        cps.append(pltpu.async_copy(
            qt_hbm.at[idx_q.at[j]], rows_q.at[pl.ds(j * CH, CH)], sem))
    for cp in cps:
        cp.wait()

    wu0 = wv[0, :]
    wu1 = wv[1, :]
    wq0 = wv[2, :]
    wq1 = wv[3, :]
    dbv = wv[4, :]
    lanes = lax.iota(jnp.int32, L)
    masks = [((lanes >> j) & 1) == 1 for j in range(4)]
    perms = [lanes ^ (1 << j) for j in range(4)]

    def group(g, carry):
        vs = []
        for r in range(L):
            row = g * L + r
            vs.append(rows_u[row, pl.ds(0, L)] * wu0
                      + rows_u[row, pl.ds(L, L)] * wu1
                      + rows_q[row, pl.ds(0, L)] * wq0
                      + rows_q[row, pl.ds(L, L)] * wq1)
        j = 0
        while len(vs) > 1:
            nxt = []
            for i in range(len(vs) // 2):
                a, b = vs[2 * i], vs[2 * i + 1]
                ab = jnp.where(masks[j], b, a)
                ba = jnp.where(masks[j], a, b)
                nxt.append(ab + ba.at[perms[j]].get(
                    mode="promise_in_bounds"))
            vs = nxt
            j += 1
        d = vs[0] + dbv
        p0 = 1.0 / (1.0 + jnp.exp(-d))
        p1 = 1.0 - p0
        half = lanes >> 1
        even = (lanes & 1) == 0
        lo0 = p0.at[half].get(mode="promise_in_bounds")
        lo1 = p1.at[half].get(mode="promise_in_bounds")
        hi0 = p0.at[half + 8].get(mode="promise_in_bounds")
        hi1 = p1.at[half + 8].get(mode="promise_in_bounds")
        outbuf[pl.ds(g * 2 * L, L)] = jnp.where(even, lo0, lo1)
        outbuf[pl.ds(g * 2 * L + L, L)] = jnp.where(even, hi0, hi1)
        return carry

    lax.fori_loop(0, GROUPS, group, 0, unroll=False)

    pltpu.sync_copy(outbuf, out_hbm.at[pl.ds(wid * RPW * 2, RPW * 2)])


@jax.jit
def _run(uid2d, qid2d, user_table, question_table, wpk):
    mesh = plsc.VectorSubcoreMesh(core_axis_name="c", subcore_axis_name="s")
    flat = pl.kernel(
        _sc_body,
        mesh=mesh,
        out_type=jax.ShapeDtypeStruct((B * 2,), jnp.float32),
        compiler_params=pltpu.CompilerParams(use_tc_tiling_on_sc=False),
        scratch_types=[
            pltpu.VMEM((NCH, CH), jnp.int32),
            pltpu.VMEM((NCH, CH), jnp.int32),
            pltpu.VMEM((RPW, D), jnp.float32),
            pltpu.VMEM((RPW, D), jnp.float32),
            pltpu.VMEM((5, L), jnp.float32),
            pltpu.VMEM((RPW * 2,), jnp.float32),
            pltpu.SemaphoreType.DMA,
        ],
    )(uid2d, qid2d, user_table, question_table, wpk)
    return flat.reshape(B, 2)


def kernel(user_ids, question_ids, user_table, question_table, W, b):
    uid2d = user_ids.astype(jnp.int32).reshape(NW * NCH, CH)
    qid2d = question_ids.astype(jnp.int32).reshape(NW * NCH, CH)
    wd = W[0] - W[1]
    db = b[0] - b[1]
    wpk = jnp.concatenate([wd, jnp.full((L,), db, jnp.float32)]).reshape(5, L)
    return _run(uid2d, qid2d, user_table, question_table, wpk)
